# Initial kernel scaffold; baseline (speedup 1.0000x reference)
import jax
import jax.numpy as jnp
from jax.experimental import pallas as pl


def kernel(inputs, target, params):
    raise NotImplementedError("write your pallas kernel here")


# speedup vs baseline: 5.3481x; 5.3481x over previous
import functools

import jax
import jax.numpy as jnp
from jax import lax
from jax.experimental import pallas as pl
from jax.experimental.pallas import tpu as pltpu
from jax.experimental.pallas import tpu_sc as plsc

V = 100000
EMB = 128
HID = 256
B = 32
L = 400
TM1 = 7
R = B * TM1

NC = 2
NS = 16
NW = NC * NS

VT = 2048
NVT = (V + VT - 1) // VT


def _sc_gather(table, idx):
    n = idx.shape[0]
    d = table.shape[1]
    b_per_w = n // NW
    mesh = plsc.VectorSubcoreMesh(core_axis_name="c", subcore_axis_name="s")

    @functools.partial(
        pl.kernel,
        mesh=mesh,
        out_type=jax.ShapeDtypeStruct((n, d), jnp.float32),
        scratch_types=[
            pltpu.VMEM((b_per_w,), jnp.int32),
            pltpu.VMEM((b_per_w, d), jnp.float32),
            pltpu.SemaphoreType.DMA,
        ],
    )
    def k(table_hbm, idx_hbm, out_hbm, idx_v, rows_v, sem):
        wid = lax.axis_index("s") * NC + lax.axis_index("c")
        base = wid * b_per_w
        pltpu.sync_copy(idx_hbm.at[pl.ds(base, b_per_w)], idx_v)
        pltpu.async_copy(table_hbm.at[idx_v], rows_v, sem).wait()
        pltpu.sync_copy(rows_v, out_hbm.at[pl.ds(base, b_per_w)])

    return k(table, idx)


def _sc_copy_scatter(inputs_idx, pc_rows):
    mesh = plsc.VectorSubcoreMesh(core_axis_name="c", subcore_axis_name="s")

    @functools.partial(
        pl.kernel,
        mesh=mesh,
        out_type=jax.ShapeDtypeStruct((R, V), jnp.float32),
        compiler_params=pltpu.CompilerParams(needs_layout_passes=False),
        scratch_types=[
            pltpu.VMEM((L,), jnp.int32),
            pltpu.VMEM((L,), jnp.float32),
            pltpu.VMEM((V,), jnp.float32),
        ],
    )
    def k(idx_hbm, pc_hbm, out_hbm, idx_v, val_v, buf_v):
        w = lax.axis_index("s") * NC + lax.axis_index("c")
        pltpu.sync_copy(idx_hbm.at[w], idx_v)

        def zero_body(i, _):
            buf_v[pl.ds(i * 16, 16)] = jnp.zeros((16,), jnp.float32)
            return 0

        lax.fori_loop(0, V // 16, zero_body, 0)

        zeros16 = jnp.zeros((16,), jnp.float32)
        for t in range(TM1):
            r = w * TM1 + t
            pltpu.sync_copy(pc_hbm.at[r], val_v)

            def add_body(j, _):
                iv = idx_v[pl.ds(j * 16, 16)]
                vv = val_v[pl.ds(j * 16, 16)]
                plsc.addupdate_scatter(buf_v, [iv], vv)
                return 0

            lax.fori_loop(0, L // 16, add_body, 0)
            pltpu.sync_copy(buf_v, out_hbm.at[r])
            if t < TM1 - 1:
                def clr_body(j, _):
                    iv = idx_v[pl.ds(j * 16, 16)]
                    plsc.store_scatter(buf_v, [iv], zeros16)
                    return 0

                lax.fori_loop(0, L // 16, clr_body, 0)

    return k(inputs_idx, pc_rows)


def _lstm_gates(x, h, c, wihT, whhT, bias):
    g = (jnp.dot(x.astype(jnp.bfloat16), wihT,
                 preferred_element_type=jnp.float32)
         + jnp.dot(h.astype(jnp.bfloat16), whhT,
                   preferred_element_type=jnp.float32) + bias)
    i = jax.nn.sigmoid(g[:, :HID])
    f = jax.nn.sigmoid(g[:, HID:2 * HID])
    gg = jnp.tanh(g[:, 2 * HID:3 * HID])
    o = jax.nn.sigmoid(g[:, 3 * HID:])
    c2 = f * c + i * gg
    h2 = o * jnp.tanh(c2)
    return h2, c2


def _enc_body(embf, embb, wihf, whhf, bf, wihb, whhb, bb, hf_out, hb_out,
              hf_s, cf_s, hb_s, cb_s):
    t = pl.program_id(0)

    @pl.when(t == 0)
    def _():
        z = jnp.zeros((B, HID), jnp.float32)
        hf_s[...] = z
        cf_s[...] = z
        hb_s[...] = z
        cb_s[...] = z

    hf, cf = _lstm_gates(embf[0], hf_s[...], cf_s[...], wihf[...], whhf[...],
                         bf[...])
    hf_s[...] = hf
    cf_s[...] = cf
    hf_out[0] = hf.astype(jnp.bfloat16)
    hb, cb = _lstm_gates(embb[0], hb_s[...], cb_s[...], wihb[...], whhb[...],
                         bb[...])
    hb_s[...] = hb
    cb_s[...] = cb
    hb_out[0] = hb.astype(jnp.bfloat16)


def _encoder(emb_enc, wihf, whhf, bf, wihb, whhb, bb):
    H4 = 4 * HID
    full = lambda shape: pl.BlockSpec(shape, lambda t: (0,) * len(shape))
    return pl.pallas_call(
        _enc_body,
        grid=(L,),
        in_specs=[
            pl.BlockSpec((1, B, EMB), lambda t: (t, 0, 0)),
            pl.BlockSpec((1, B, EMB), lambda t: (L - 1 - t, 0, 0)),
            full((EMB, H4)), full((HID, H4)), full((1, H4)),
            full((EMB, H4)), full((HID, H4)), full((1, H4)),
        ],
        out_specs=[
            pl.BlockSpec((1, B, HID), lambda t: (t, 0, 0)),
            pl.BlockSpec((1, B, HID), lambda t: (L - 1 - t, 0, 0)),
        ],
        out_shape=[
            jax.ShapeDtypeStruct((L, B, HID), jnp.bfloat16),
            jax.ShapeDtypeStruct((L, B, HID), jnp.bfloat16),
        ],
        scratch_shapes=[pltpu.VMEM((B, HID), jnp.float32)] * 4,
    )(emb_enc, emb_enc, wihf, whhf, bf, wihb, whhb, bb)


def _dec_body(hf, hb, emb_dec, wih, whh, bdec, v_w, wh_w_mat, wh_b_vec,
              ws_mat, ws_b_vec, v_b, pm, v1wT, v1b, whw, wsw, wxw, pgb,
              pc_out, hid1_out, pg_out, cov_out):
    h = jnp.zeros((B, HID), jnp.float32)
    c = jnp.zeros((B, HID), jnp.float32)
    h2s = []
    for t in range(TM1):
        h, c = _lstm_gates(emb_dec[t], h, c, wih[...], whh[...], bdec[...])
        h2s.append(h)
    h2_all = jnp.stack(h2s, axis=1)

    w_enc = jnp.dot(v_w[...], wh_w_mat[...],
                    preferred_element_type=jnp.float32)[0]
    c_enc = jnp.sum(wh_b_vec[...] * v_w[...]) + v_b[0, 0]
    ws_w = jnp.dot(v_w[...], ws_mat[...],
                   preferred_element_type=jnp.float32)[0]
    ws_c = jnp.sum(ws_b_vec[...] * v_w[...])
    hfr = hf[...].reshape(L * B, HID)
    hbr = hb[...].reshape(L * B, HID)
    enc_dot = (jnp.sum(hfr * w_enc[None, :HID], axis=1)
               + jnp.sum(hbr * w_enc[None, HID:], axis=1))
    enc_dot = enc_dot.reshape(L, B).T + c_enc
    ws_dot = jnp.sum(h2_all * ws_w[None, None, :], axis=2) + ws_c
    eo = jnp.dot(ws_dot.T, pm[...],
                 preferred_element_type=jnp.float32)
    par_b = (lax.broadcasted_iota(jnp.int32, (B, 1, 1), 0) % 2) == 0
    ws_sel = jnp.where(par_b, eo[None, :, :L], eo[None, :, L:])
    a = enc_dot[:, None, :] + ws_sel
    a = a - jnp.max(a, axis=2, keepdims=True)
    e = jnp.exp(a)
    attn = e / jnp.sum(e, axis=2, keepdims=True)

    cov = jnp.zeros((B, L), jnp.float32)
    loss = jnp.float32(0.0)
    for t in range(TM1):
        at = attn[:, t, :]
        loss = loss + jnp.sum(jnp.minimum(at, cov))
        cov = cov + at
    cov_out[0, 0] = loss

    attn2d = attn.reshape(R, L).astype(jnp.bfloat16)
    yf = jnp.dot(attn2d, hf[...].reshape(L, B * HID),
                 preferred_element_type=jnp.float32).reshape(B, TM1, B, HID)
    yb = jnp.dot(attn2d, hb[...].reshape(L, B * HID),
                 preferred_element_type=jnp.float32).reshape(B, TM1, B, HID)
    ctxf = jnp.stack([yf[b, :, b, :] for b in range(B)], axis=0)
    ctxb = jnp.stack([yb[b, :, b, :] for b in range(B)], axis=0)

    cat = jnp.concatenate([h2_all, ctxf, ctxb], axis=2)
    hid1 = (jnp.dot(cat.reshape(R, 3 * HID).astype(jnp.bfloat16), v1wT[...],
                    preferred_element_type=jnp.float32) + v1b[...])
    hid1_out[...] = hid1.astype(jnp.bfloat16)

    xall = jnp.stack([emb_dec[t] for t in range(TM1)], axis=1)
    whw_v, wsw_v, wxw_v = whw[...], wsw[...], wxw[...]
    pg_lin = (jnp.sum(ctxf * whw_v[None, None, :HID], axis=2)
              + jnp.sum(ctxb * whw_v[None, None, HID:], axis=2)
              + jnp.sum(h2_all * wsw_v[None, None, :], axis=2)
              + jnp.sum(xall * wxw_v[None, None, :], axis=2) + pgb[0, 0])
    pg = jax.nn.sigmoid(pg_lin)
    pg_out[...] = pg
    pc_out[...] = (1.0 - pg)[:, :, None] * attn


def _decoder(hf, hb, emb_dec, wih, whh, bdec, v_w, wh_w_mat, wh_b_vec,
             ws_mat, ws_b_vec, v_b, pm, v1wT, v1b, whw, wsw, wxw, pgb):
    H4 = 4 * HID
    specs = [pl.BlockSpec(x.shape, lambda *_, n=x.ndim: (0,) * n) for x in
             (hf, hb, emb_dec, wih, whh, bdec, v_w, wh_w_mat, wh_b_vec,
              ws_mat, ws_b_vec, v_b, pm, v1wT, v1b, whw, wsw, wxw, pgb)]
    return pl.pallas_call(
        _dec_body,
        grid=(1,),
        in_specs=specs,
        out_specs=[
            pl.BlockSpec((B, TM1, L), lambda *_: (0, 0, 0)),
            pl.BlockSpec((R, 3 * HID), lambda *_: (0, 0)),
            pl.BlockSpec((B, TM1), lambda *_: (0, 0)),
            pl.BlockSpec((1, 1), lambda *_: (0, 0),
                         memory_space=pltpu.SMEM),
        ],
        out_shape=[
            jax.ShapeDtypeStruct((B, TM1, L), jnp.float32),
            jax.ShapeDtypeStruct((R, 3 * HID), jnp.bfloat16),
            jax.ShapeDtypeStruct((B, TM1), jnp.float32),
            jax.ShapeDtypeStruct((1, 1), jnp.float32),
        ],
    )(hf, hb, emb_dec, wih, whh, bdec, v_w, wh_w_mat, wh_b_vec, ws_mat,
      ws_b_vec, v_b, pm, v1wT, v1b, whw, wsw, wxw, pgb)


def _logits_body(hid1, v2, v2b, lg_out, m_out, s_out, m_s, s_s):
    i = pl.program_id(0)

    @pl.when(i == 0)
    def _():
        m_s[...] = jnp.full((R, 1), -1e30, jnp.float32)
        s_s[...] = jnp.zeros((R, 1), jnp.float32)

    lg = lax.dot_general(hid1[...], v2[...].astype(jnp.bfloat16),
                         (((1,), (1,)), ((), ())),
                         preferred_element_type=jnp.float32) + v2b[...][None, :]
    lg_out[...] = lg
    col = lax.broadcasted_iota(jnp.int32, (R, VT), 1) + i * VT
    lgm = jnp.where(col < V, lg, -1e30)
    tile_m = jnp.max(lgm, axis=1, keepdims=True)
    new_m = jnp.maximum(m_s[...], tile_m)
    s_s[...] = (s_s[...] * jnp.exp(m_s[...] - new_m)
                + jnp.sum(jnp.exp(lgm - new_m), axis=1, keepdims=True))
    m_s[...] = new_m
    m_out[...] = new_m
    s_out[...] = s_s[...]


def _logits(hid1, v2w, v2b):
    return pl.pallas_call(
        _logits_body,
        grid=(NVT,),
        in_specs=[
            pl.BlockSpec((R, 3 * HID), lambda i: (0, 0)),
            pl.BlockSpec((VT, 3 * HID), lambda i: (i, 0)),
            pl.BlockSpec((VT,), lambda i: (i,)),
        ],
        out_specs=[
            pl.BlockSpec((R, VT), lambda i: (0, i)),
            pl.BlockSpec((R, 1), lambda i: (0, 0)),
            pl.BlockSpec((R, 1), lambda i: (0, 0)),
        ],
        out_shape=[
            jax.ShapeDtypeStruct((R, V), jnp.float32),
            jax.ShapeDtypeStruct((R, 1), jnp.float32),
            jax.ShapeDtypeStruct((R, 1), jnp.float32),
        ],
        scratch_shapes=[pltpu.VMEM((R, 1), jnp.float32)] * 2,
    )(hid1, v2w, v2b)


def _combine_body(lg, cp, m, s, pg, out):
    pv = jnp.exp(lg[...] - m[...]) / s[...]
    out[...] = pg[...] * pv + cp[...] + jnp.float32(1.0 / V)


def _combine(logits, copy, m, s, pg):
    return pl.pallas_call(
        _combine_body,
        grid=(NVT,),
        in_specs=[
            pl.BlockSpec((R, VT), lambda i: (0, i)),
            pl.BlockSpec((R, VT), lambda i: (0, i)),
            pl.BlockSpec((R, 1), lambda i: (0, 0)),
            pl.BlockSpec((R, 1), lambda i: (0, 0)),
            pl.BlockSpec((R, 1), lambda i: (0, 0)),
        ],
        out_specs=pl.BlockSpec((R, VT), lambda i: (0, i)),
        out_shape=jax.ShapeDtypeStruct((R, V), jnp.float32),
    )(logits, copy, m, s, pg)


def _perm_matrix():
    import numpy as np
    lidx = np.arange(L)
    pm = np.zeros((B, 2 * L), np.float32)
    for par in range(2):
        pm[(16 * par + lidx) % 32, par * L + lidx] = 1.0
    return jnp.asarray(pm)


def kernel(inputs, target, params):
    p = params
    idx_enc = jnp.minimum(inputs, V - 1).T.reshape(-1)
    idx_dec = target[:, :TM1].T.reshape(-1)
    pad = (-(idx_enc.shape[0] + idx_dec.shape[0])) % (8 * NW)
    idx_all = jnp.concatenate(
        [idx_enc, idx_dec, jnp.zeros((pad,), jnp.int32)])
    emb_all = _sc_gather(p['embed_W'], idx_all)
    emb_enc = emb_all[:L * B].reshape(L, B, EMB)
    emb_dec = emb_all[L * B:L * B + TM1 * B].reshape(TM1, B, EMB)

    bf16 = jnp.bfloat16
    hf, hb = _encoder(
        emb_enc,
        p['enc_f_Wih'].T.astype(bf16), p['enc_f_Whh'].T.astype(bf16),
        (p['enc_f_bih'] + p['enc_f_bhh']).reshape(1, -1),
        p['enc_b_Wih'].T.astype(bf16), p['enc_b_Whh'].T.astype(bf16),
        (p['enc_b_bih'] + p['enc_b_bhh']).reshape(1, -1))

    pgb = (p['wh_b'][0] + p['ws_b'][0] + p['wx_b'][0]).reshape(1, 1)
    pc_attn, hid1, pg, cov = _decoder(
        hf, hb, emb_dec,
        p['dec_Wih'].T.astype(bf16), p['dec_Whh'].T.astype(bf16),
        (p['dec_bih'] + p['dec_bhh']).reshape(1, -1),
        p['v_W'], p['Wh_W'], p['Wh_b'].reshape(1, -1),
        p['Ws_W'], p['Ws_b'].reshape(1, -1), p['v_b'].reshape(1, 1),
        _perm_matrix(), p['V1_W'].T.astype(bf16),
        p['V1_b'].reshape(1, -1),
        p['wh_W'][0], p['ws_W'][0], p['wx_W'][0], pgb)

    logits, m, s = _logits(hid1, p['V2_W'], p['V2_b'])
    copy = _sc_copy_scatter(inputs, pc_attn.reshape(R, L))
    p_w = _combine(logits, copy, m, s, pg.reshape(R, 1))
    return p_w.reshape(B, TM1, V), cov.reshape(())
